# baseline (speedup 1.0000x reference)
import functools

import jax
import jax.numpy as jnp
from jax import lax
from jax.experimental import pallas as pl
from jax.experimental.pallas import tpu as pltpu
from jax.experimental.pallas import tpu_sc as plsc

N_NODES = 10000
F_IN = 500
N_CLS = 3

NPAD = 10240
FPAD = 512
W1TAB = 80
W2TAB = 16

ETOT = 160000 + N_NODES
NTILES = 32
EB = 128
CE = 5376
EPAD = NTILES * CE
NBLK = CE // EB
STRIPE = NPAD // 16


def _stage1_body(x_ref, w1_ref, wsd_ref, t1_ref, asad_ref):
    hb = jnp.dot(x_ref[...], w1_ref[...], preferred_element_type=jnp.float32)
    col = lax.broadcasted_iota(jnp.int32, (1, W1TAB), 1)
    ones_cols = jnp.where((col >= 64) & (col < 72), 1.0, 0.0)
    t1_ref[...] = jnp.pad(hb, ((0, 0), (0, W1TAB - 64))) + ones_cols
    asad_ref[...] = jnp.dot(hb, wsd_ref[...], preferred_element_type=jnp.float32)


def _stage1(x, w1, wsd):
    grid = NPAD // 256
    return pl.pallas_call(
        _stage1_body,
        grid=(grid,),
        in_specs=[
            pl.BlockSpec((256, F_IN), lambda i: (i, 0)),
            pl.BlockSpec((F_IN, 64), lambda i: (0, 0)),
            pl.BlockSpec((64, 16), lambda i: (0, 0)),
        ],
        out_specs=[
            pl.BlockSpec((256, W1TAB), lambda i: (i, 0)),
            pl.BlockSpec((256, 16), lambda i: (i, 0)),
        ],
        out_shape=[
            jax.ShapeDtypeStruct((NPAD, W1TAB), jnp.float32),
            jax.ShapeDtypeStruct((NPAD, 16), jnp.float32),
        ],
    )(x, w1, wsd)


def _stage2_body(a_ref, b_ref, r_ref, b1_ref, wc2_ref, t2_ref, as_ref, ad_ref):
    acc = a_ref[...] + b_ref[...]
    num = acc[:, :64]
    den = acc[:, 64:72]
    denrep = jnp.dot(den, r_ref[...], preferred_element_type=jnp.float32)
    out1 = num / (denrep + 1e-16) + b1_ref[...]
    hx = jnp.where(out1 > 0, out1, jnp.exp(out1) - 1.0)
    h2x3 = jnp.dot(hx, wc2_ref[...], preferred_element_type=jnp.float32)
    col = lax.broadcasted_iota(jnp.int32, (1, 16), 1)
    t2_ref[...] = h2x3[:, :16] + jnp.where(col == 3, 1.0, 0.0)
    as_ref[...] = h2x3[:, 16:32]
    ad_ref[...] = h2x3[:, 32:48]


def _stage2(acc1a, acc1b, rmat, b1row, wc2):
    grid = NPAD // 256
    spec16 = pl.BlockSpec((256, 16), lambda i: (i, 0))
    shp16 = jax.ShapeDtypeStruct((NPAD, 16), jnp.float32)
    return pl.pallas_call(
        _stage2_body,
        grid=(grid,),
        in_specs=[
            pl.BlockSpec((256, W1TAB), lambda i: (i, 0)),
            pl.BlockSpec((256, W1TAB), lambda i: (i, 0)),
            pl.BlockSpec((8, 64), lambda i: (0, 0)),
            pl.BlockSpec((1, 64), lambda i: (0, 0)),
            pl.BlockSpec((64, 48), lambda i: (0, 0)),
        ],
        out_specs=[spec16, spec16, spec16],
        out_shape=[shp16, shp16, shp16],
    )(acc1a, acc1b, rmat, b1row, wc2)


def _stage3_body(a_ref, b_ref, b3_ref, b2_ref, out_ref, lp_ref):
    acc = a_ref[...] + b_ref[...]
    den = jnp.dot(acc, b3_ref[...], preferred_element_type=jnp.float32)
    o = acc / (den + 1e-16) + b2_ref[...]
    col = lax.broadcasted_iota(jnp.int32, (1, 16), 1)
    z = jnp.where(col < 3, o, -1e30)
    m = jnp.max(z, axis=1, keepdims=True)
    s = jnp.sum(jnp.exp(z - m), axis=1, keepdims=True)
    lp_ref[...] = z - (m + jnp.log(s))
    out_ref[...] = o


def _stage3(acc2a, acc2b, b3mat, b2row):
    grid = NPAD // 256
    spec16 = pl.BlockSpec((256, 16), lambda i: (i, 0))
    shp16 = jax.ShapeDtypeStruct((NPAD, 16), jnp.float32)
    return pl.pallas_call(
        _stage3_body,
        grid=(grid,),
        in_specs=[
            spec16,
            spec16,
            pl.BlockSpec((16, 16), lambda i: (0, 0)),
            pl.BlockSpec((1, 16), lambda i: (0, 0)),
        ],
        out_specs=[spec16, spec16],
        out_shape=[shp16, shp16],
    )(acc2a, acc2b, b3mat, b2row)


def _vexpand(vec, idx):
    dn = lax.GatherDimensionNumbers(
        offset_dims=(), collapsed_slice_dims=(0,), start_index_map=(0,))
    return lax.gather(vec, idx[:, None], dn, (1,),
                      mode=lax.GatherScatterMode.PROMISE_IN_BOUNDS)


def _edge_body(w, compact, t_hbm, as_hbm, ad_hbm, src_hbm, dst_hbm,
               out0_hbm, out1_hbm,
               sidx0, sidx1, didx0, didx1, t0, t1b, a0, a1, d0, d1, acc,
               st0, st1, sa0, sa1, sd0, sd1):
    kw = w // 16
    c = lax.axis_index("c")
    s = lax.axis_index("s")
    wid = s * 2 + c
    ebase = wid * CE

    sidx = (sidx0, sidx1)
    didx = (didx0, didx1)
    trows = (t0, t1b)
    asrows = (a0, a1)
    adrows = (d0, d1)
    sem_t = (st0, st1)
    sem_a = (sa0, sa1)
    sem_d = (sd0, sd1)

    def _zrow(r, carry):
        for k in range(kw):
            t0[r, pl.ds(k * 16, 16)] = jnp.zeros((16,), jnp.float32)
        return carry
    lax.fori_loop(0, EB, _zrow, 0)
    for j in range(STRIPE // EB):
        pltpu.sync_copy(t0, acc.at[pl.ds(s * STRIPE + j * EB, EB), :])
    plsc.subcore_barrier()

    def _fetch(b, buf):
        base = ebase + b * EB
        pltpu.sync_copy(src_hbm.at[pl.ds(base, EB)], sidx[buf])
        pltpu.sync_copy(dst_hbm.at[pl.ds(base, EB)], didx[buf])
        pltpu.async_copy(t_hbm.at[sidx[buf]], trows[buf], sem_t[buf])
        pltpu.async_copy(as_hbm.at[sidx[buf]], asrows[buf], sem_a[buf])
        pltpu.async_copy(ad_hbm.at[didx[buf]], adrows[buf], sem_d[buf])

    def _wait(buf):
        pltpu.make_async_copy(t_hbm.at[sidx[buf]], trows[buf], sem_t[buf]).wait()
        pltpu.make_async_copy(as_hbm.at[sidx[buf]], asrows[buf], sem_a[buf]).wait()
        pltpu.make_async_copy(ad_hbm.at[didx[buf]], adrows[buf], sem_d[buf]).wait()

    _fetch(0, 0)

    def _outer(i, carry):
        for j in range(2):
            b = 2 * i + j
            nb = b + 1

            @pl.when(nb < NBLK)
            def _():
                _fetch(nb, j ^ 1)

            _wait(j)

            if compact:
                iota = lax.iota(jnp.int32, 16)
                shift_idx = (iota & 7) + 8
                widx = [jnp.where(iota < 8, 2 * k, 2 * k + 1) for k in range(4)]

                def _one(r):
                    sreg = asrows[j][r, :]
                    dreg = adrows[j][r, :]
                    e = sreg + _vexpand(dreg, shift_idx)
                    e = jnp.where(e >= 0, e, 0.2 * e)
                    wv = jnp.exp(e)
                    for k in range(4):
                        sl = pl.ds(k * 16, 16)
                        trows[j][r, sl] = trows[j][r, sl] * _vexpand(wv, widx[k])
                    sl = pl.ds(64, 16)
                    trows[j][r, sl] = trows[j][r, sl] * wv

            else:
                def _one(r):
                    for k in range(kw):
                        sl = pl.ds(k * 16, 16)
                        e = asrows[j][r, sl] + adrows[j][r, sl]
                        e = jnp.where(e >= 0, e, 0.2 * e)
                        trows[j][r, sl] = trows[j][r, sl] * jnp.exp(e)

            @plsc.parallel_loop(0, EB, unroll=4)
            def _rows(r):
                _one(r)

            pltpu.sync_copy(trows[j], acc.at[didx[j]], add=True)
        return carry
    lax.fori_loop(0, NBLK // 2, _outer, 0)

    plsc.subcore_barrier()
    rows = pl.ds(s * STRIPE, STRIPE)

    @pl.when(c == 0)
    def _():
        pltpu.sync_copy(acc.at[rows, :], out0_hbm.at[rows, :])

    @pl.when(c == 1)
    def _():
        pltpu.sync_copy(acc.at[rows, :], out1_hbm.at[rows, :])


def _edge_pass(w, t_tab, as_tab, ad_tab, src, dst, compact=False):
    mesh = plsc.VectorSubcoreMesh(core_axis_name="c", subcore_axis_name="s")
    shp = jax.ShapeDtypeStruct((NPAD, w), jnp.float32)
    wa = 16 if compact else w
    kern = functools.partial(
        pl.kernel,
        out_type=(shp, shp),
        mesh=mesh,
        compiler_params=pltpu.CompilerParams(use_tc_tiling_on_sc=False),
        scratch_types=[
            pltpu.VMEM((EB,), jnp.int32),
            pltpu.VMEM((EB,), jnp.int32),
            pltpu.VMEM((EB,), jnp.int32),
            pltpu.VMEM((EB,), jnp.int32),
            pltpu.VMEM((EB, w), jnp.float32),
            pltpu.VMEM((EB, w), jnp.float32),
            pltpu.VMEM((EB, wa), jnp.float32),
            pltpu.VMEM((EB, wa), jnp.float32),
            pltpu.VMEM((EB, wa), jnp.float32),
            pltpu.VMEM((EB, wa), jnp.float32),
            pltpu.VMEM_SHARED((NPAD, w), jnp.float32),
            pltpu.SemaphoreType.DMA,
            pltpu.SemaphoreType.DMA,
            pltpu.SemaphoreType.DMA,
            pltpu.SemaphoreType.DMA,
            pltpu.SemaphoreType.DMA,
            pltpu.SemaphoreType.DMA,
        ],
    )(functools.partial(_edge_body, w, compact))
    return kern(t_tab, as_tab, ad_tab, src, dst)


def kernel(x, edge_index, W1, att_src1, att_dst1, b1, W2, att_src2, att_dst2, b2):
    f32 = jnp.float32
    eye8 = jnp.eye(8, dtype=f32)

    ms = (att_src1[0][:, :, None] * eye8[:, None, :]).reshape(64, 8)
    md = (att_dst1[0][:, :, None] * eye8[:, None, :]).reshape(64, 8)
    rmat = (eye8[:, :, None] * jnp.ones((1, 1, 8), f32)).reshape(8, 64)
    wsd = jnp.concatenate([ms, md], axis=1)

    w2p = jnp.pad(W2, ((0, 0), (0, 16 - N_CLS)))
    col16 = jnp.arange(16)
    msel = jnp.where((col16[None, :] < 4) & (col16[:, None] < 3),
                     1.0, 0.0).astype(f32)
    m_s = msel * jnp.pad(att_src2[0, 0], (0, 13))[:, None]
    m_d = msel * jnp.pad(att_dst2[0, 0], (0, 13))[:, None]
    wc2 = jnp.concatenate([w2p, w2p @ m_s, w2p @ m_d], axis=1)

    b1row = b1.reshape(1, 64)
    b3mat = jnp.where((jnp.arange(16)[:, None] == 3), 1.0, 0.0
                      ).astype(f32) * jnp.ones((1, 16), f32)
    b2row = jnp.pad(b2, (0, 16 - N_CLS)).reshape(1, 16)

    loop = jnp.arange(N_NODES, dtype=edge_index.dtype)
    padv = jnp.full((EPAD - ETOT,), N_NODES, dtype=edge_index.dtype)
    src = jnp.concatenate([edge_index[0], loop, padv])
    dst = jnp.concatenate([edge_index[1], loop, padv])

    t1, asad = _stage1(x, W1, wsd)

    acc1a, acc1b = _edge_pass(W1TAB, t1, asad, asad, src, dst, compact=True)

    t2, as16, ad16 = _stage2(acc1a, acc1b, rmat, b1row, wc2)

    acc2a, acc2b = _edge_pass(W2TAB, t2, as16, ad16, src, dst)

    out16, lp16 = _stage3(acc2a, acc2b, b3mat, b2row)
    return (out16[:N_NODES, :N_CLS], lp16[:N_NODES, :N_CLS])


# speedup vs baseline: 1.0786x; 1.0786x over previous
import functools

import jax
import jax.numpy as jnp
from jax import lax
from jax.experimental import pallas as pl
from jax.experimental.pallas import tpu as pltpu
from jax.experimental.pallas import tpu_sc as plsc

N_NODES = 10000
F_IN = 500
N_CLS = 3

NPAD = 10240
FPAD = 512
W1TAB = 80
W2TAB = 16

ETOT = 160000 + N_NODES
NTILES = 32
EB = 128
CE = 5376
EPAD = NTILES * CE
NBLK = CE // EB
STRIPE = NPAD // 16


def _stage1_body(x_ref, w1_ref, wsd_ref, t1_ref, asad_ref):
    hb = jnp.dot(x_ref[...], w1_ref[...], preferred_element_type=jnp.float32)
    col = lax.broadcasted_iota(jnp.int32, (1, W1TAB), 1)
    ones_cols = jnp.where((col >= 64) & (col < 72), 1.0, 0.0)
    t1_ref[...] = jnp.pad(hb, ((0, 0), (0, W1TAB - 64))) + ones_cols
    asad_ref[...] = jnp.dot(hb, wsd_ref[...], preferred_element_type=jnp.float32)


def _stage1(x, w1, wsd):
    grid = NPAD // 256
    return pl.pallas_call(
        _stage1_body,
        grid=(grid,),
        in_specs=[
            pl.BlockSpec((256, F_IN), lambda i: (i, 0)),
            pl.BlockSpec((F_IN, 64), lambda i: (0, 0)),
            pl.BlockSpec((64, 16), lambda i: (0, 0)),
        ],
        out_specs=[
            pl.BlockSpec((256, W1TAB), lambda i: (i, 0)),
            pl.BlockSpec((256, 16), lambda i: (i, 0)),
        ],
        out_shape=[
            jax.ShapeDtypeStruct((NPAD, W1TAB), jnp.float32),
            jax.ShapeDtypeStruct((NPAD, 16), jnp.float32),
        ],
    )(x, w1, wsd)


def _stage2_body(a_ref, b_ref, r_ref, b1_ref, wc2_ref, t2_ref, as_ref, ad_ref):
    acc = a_ref[...] + b_ref[...]
    num = acc[:, :64]
    den = acc[:, 64:72]
    denrep = jnp.dot(den, r_ref[...], preferred_element_type=jnp.float32)
    out1 = num / (denrep + 1e-16) + b1_ref[...]
    hx = jnp.where(out1 > 0, out1, jnp.exp(out1) - 1.0)
    h2x3 = jnp.dot(hx, wc2_ref[...], preferred_element_type=jnp.float32)
    col = lax.broadcasted_iota(jnp.int32, (1, 16), 1)
    t2_ref[...] = h2x3[:, :16] + jnp.where(col == 3, 1.0, 0.0)
    as_ref[...] = h2x3[:, 16:32]
    ad_ref[...] = h2x3[:, 32:48]


def _stage2(acc1a, acc1b, rmat, b1row, wc2):
    grid = NPAD // 256
    spec16 = pl.BlockSpec((256, 16), lambda i: (i, 0))
    shp16 = jax.ShapeDtypeStruct((NPAD, 16), jnp.float32)
    return pl.pallas_call(
        _stage2_body,
        grid=(grid,),
        in_specs=[
            pl.BlockSpec((256, W1TAB), lambda i: (i, 0)),
            pl.BlockSpec((256, W1TAB), lambda i: (i, 0)),
            pl.BlockSpec((8, 64), lambda i: (0, 0)),
            pl.BlockSpec((1, 64), lambda i: (0, 0)),
            pl.BlockSpec((64, 48), lambda i: (0, 0)),
        ],
        out_specs=[spec16, spec16, spec16],
        out_shape=[shp16, shp16, shp16],
    )(acc1a, acc1b, rmat, b1row, wc2)


def _stage3_body(a_ref, b_ref, b3_ref, b2_ref, out_ref, lp_ref):
    acc = a_ref[...] + b_ref[...]
    den = jnp.dot(acc, b3_ref[...], preferred_element_type=jnp.float32)
    o = acc / (den + 1e-16) + b2_ref[...]
    col = lax.broadcasted_iota(jnp.int32, (1, 16), 1)
    z = jnp.where(col < 3, o, -1e30)
    m = jnp.max(z, axis=1, keepdims=True)
    s = jnp.sum(jnp.exp(z - m), axis=1, keepdims=True)
    lp_ref[...] = z - (m + jnp.log(s))
    out_ref[...] = o


def _stage3(acc2a, acc2b, b3mat, b2row):
    grid = NPAD // 256
    spec16 = pl.BlockSpec((256, 16), lambda i: (i, 0))
    shp16 = jax.ShapeDtypeStruct((NPAD, 16), jnp.float32)
    return pl.pallas_call(
        _stage3_body,
        grid=(grid,),
        in_specs=[
            spec16,
            spec16,
            pl.BlockSpec((16, 16), lambda i: (0, 0)),
            pl.BlockSpec((1, 16), lambda i: (0, 0)),
        ],
        out_specs=[spec16, spec16],
        out_shape=[shp16, shp16],
    )(acc2a, acc2b, b3mat, b2row)


def _vexpand(vec, idx):
    dn = lax.GatherDimensionNumbers(
        offset_dims=(), collapsed_slice_dims=(0,), start_index_map=(0,))
    return lax.gather(vec, idx[:, None], dn, (1,),
                      mode=lax.GatherScatterMode.PROMISE_IN_BOUNDS)


def _edge_body(w, compact, t_hbm, as_hbm, ad_hbm, src_hbm, dst_hbm,
               out0_hbm, out1_hbm,
               sidx0, sidx1, didx0, didx1, sall, dall,
               t0, t1b, a0, a1, d0, d1, acc,
               st0, st1, sa0, sa1, sd0, sd1):
    kw = w // 16
    c = lax.axis_index("c")
    s = lax.axis_index("s")
    wid = s * 2 + c
    ebase = wid * CE

    sidx = (sidx0, sidx1)
    didx = (didx0, didx1)
    trows = (t0, t1b)
    asrows = (a0, a1)
    adrows = (d0, d1)
    sem_t = (st0, st1)
    sem_a = (sa0, sa1)
    sem_d = (sd0, sd1)

    def _zrow(r, carry):
        for k in range(kw):
            t0[r, pl.ds(k * 16, 16)] = jnp.zeros((16,), jnp.float32)
        return carry
    lax.fori_loop(0, EB, _zrow, 0)
    for j in range(STRIPE // EB):
        pltpu.sync_copy(t0, acc.at[pl.ds(s * STRIPE + j * EB, EB), :])
    plsc.subcore_barrier()

    pltpu.sync_copy(src_hbm.at[pl.ds(ebase, CE)], sall)
    pltpu.sync_copy(dst_hbm.at[pl.ds(ebase, CE)], dall)

    def _fetch(b, buf):
        for k in range(EB // 16):
            sl = pl.ds(k * 16, 16)
            sidx[buf][sl] = sall[pl.ds(b * EB + k * 16, 16)]
            didx[buf][sl] = dall[pl.ds(b * EB + k * 16, 16)]
        pltpu.async_copy(t_hbm.at[sidx[buf]], trows[buf], sem_t[buf])
        pltpu.async_copy(as_hbm.at[sidx[buf]], asrows[buf], sem_a[buf])
        pltpu.async_copy(ad_hbm.at[didx[buf]], adrows[buf], sem_d[buf])

    def _wait(buf):
        pltpu.make_async_copy(t_hbm.at[sidx[buf]], trows[buf], sem_t[buf]).wait()
        pltpu.make_async_copy(as_hbm.at[sidx[buf]], asrows[buf], sem_a[buf]).wait()
        pltpu.make_async_copy(ad_hbm.at[didx[buf]], adrows[buf], sem_d[buf]).wait()

    _fetch(0, 0)

    def _outer(i, carry):
        for j in range(2):
            b = 2 * i + j
            nb = b + 1

            @pl.when(nb < NBLK)
            def _():
                _fetch(nb, j ^ 1)

            _wait(j)

            if compact:
                iota = lax.iota(jnp.int32, 16)
                shift_idx = (iota & 7) + 8
                widx = [jnp.where(iota < 8, 2 * k, 2 * k + 1) for k in range(4)]

                def _one(r):
                    sreg = asrows[j][r, :]
                    dreg = adrows[j][r, :]
                    e = sreg + _vexpand(dreg, shift_idx)
                    e = jnp.where(e >= 0, e, 0.2 * e)
                    wv = jnp.exp(e)
                    for k in range(4):
                        sl = pl.ds(k * 16, 16)
                        trows[j][r, sl] = trows[j][r, sl] * _vexpand(wv, widx[k])
                    sl = pl.ds(64, 16)
                    trows[j][r, sl] = trows[j][r, sl] * wv

            else:
                def _one(r):
                    for k in range(kw):
                        sl = pl.ds(k * 16, 16)
                        e = asrows[j][r, sl] + adrows[j][r, sl]
                        e = jnp.where(e >= 0, e, 0.2 * e)
                        trows[j][r, sl] = trows[j][r, sl] * jnp.exp(e)

            @plsc.parallel_loop(0, EB, unroll=2)
            def _rows(r):
                _one(r)

            pltpu.sync_copy(trows[j], acc.at[didx[j]], add=True)
        return carry
    lax.fori_loop(0, NBLK // 2, _outer, 0)

    plsc.subcore_barrier()
    rows = pl.ds(s * STRIPE, STRIPE)

    @pl.when(c == 0)
    def _():
        pltpu.sync_copy(acc.at[rows, :], out0_hbm.at[rows, :])

    @pl.when(c == 1)
    def _():
        pltpu.sync_copy(acc.at[rows, :], out1_hbm.at[rows, :])


def _edge_pass(w, t_tab, as_tab, ad_tab, src, dst, compact=False):
    mesh = plsc.VectorSubcoreMesh(core_axis_name="c", subcore_axis_name="s")
    shp = jax.ShapeDtypeStruct((NPAD, w), jnp.float32)
    wa = 16 if compact else w
    kern = functools.partial(
        pl.kernel,
        out_type=(shp, shp),
        mesh=mesh,
        compiler_params=pltpu.CompilerParams(use_tc_tiling_on_sc=False),
        scratch_types=[
            pltpu.VMEM((EB,), jnp.int32),
            pltpu.VMEM((EB,), jnp.int32),
            pltpu.VMEM((EB,), jnp.int32),
            pltpu.VMEM((EB,), jnp.int32),
            pltpu.VMEM((CE,), jnp.int32),
            pltpu.VMEM((CE,), jnp.int32),
            pltpu.VMEM((EB, w), jnp.float32),
            pltpu.VMEM((EB, w), jnp.float32),
            pltpu.VMEM((EB, wa), jnp.float32),
            pltpu.VMEM((EB, wa), jnp.float32),
            pltpu.VMEM((EB, wa), jnp.float32),
            pltpu.VMEM((EB, wa), jnp.float32),
            pltpu.VMEM_SHARED((NPAD, w), jnp.float32),
            pltpu.SemaphoreType.DMA,
            pltpu.SemaphoreType.DMA,
            pltpu.SemaphoreType.DMA,
            pltpu.SemaphoreType.DMA,
            pltpu.SemaphoreType.DMA,
            pltpu.SemaphoreType.DMA,
        ],
    )(functools.partial(_edge_body, w, compact))
    return kern(t_tab, as_tab, ad_tab, src, dst)


def kernel(x, edge_index, W1, att_src1, att_dst1, b1, W2, att_src2, att_dst2, b2):
    f32 = jnp.float32
    eye8 = jnp.eye(8, dtype=f32)

    ms = (att_src1[0][:, :, None] * eye8[:, None, :]).reshape(64, 8)
    md = (att_dst1[0][:, :, None] * eye8[:, None, :]).reshape(64, 8)
    rmat = (eye8[:, :, None] * jnp.ones((1, 1, 8), f32)).reshape(8, 64)
    wsd = jnp.concatenate([ms, md], axis=1)

    w2p = jnp.pad(W2, ((0, 0), (0, 16 - N_CLS)))
    col16 = jnp.arange(16)
    msel = jnp.where((col16[None, :] < 4) & (col16[:, None] < 3),
                     1.0, 0.0).astype(f32)
    m_s = msel * jnp.pad(att_src2[0, 0], (0, 13))[:, None]
    m_d = msel * jnp.pad(att_dst2[0, 0], (0, 13))[:, None]
    wc2 = jnp.concatenate([w2p, w2p @ m_s, w2p @ m_d], axis=1)

    b1row = b1.reshape(1, 64)
    b3mat = jnp.where((jnp.arange(16)[:, None] == 3), 1.0, 0.0
                      ).astype(f32) * jnp.ones((1, 16), f32)
    b2row = jnp.pad(b2, (0, 16 - N_CLS)).reshape(1, 16)

    loop = jnp.arange(N_NODES, dtype=edge_index.dtype)
    padv = jnp.full((EPAD - ETOT,), N_NODES, dtype=edge_index.dtype)
    src = jnp.concatenate([edge_index[0], loop, padv])
    dst = jnp.concatenate([edge_index[1], loop, padv])

    t1, asad = _stage1(x, W1, wsd)

    acc1a, acc1b = _edge_pass(W1TAB, t1, asad, asad, src, dst, compact=True)

    t2, as16, ad16 = _stage2(acc1a, acc1b, rmat, b1row, wc2)

    acc2a, acc2b = _edge_pass(W2TAB, t2, as16, ad16, src, dst)

    out16, lp16 = _stage3(acc2a, acc2b, b3mat, b2row)
    return (out16[:N_NODES, :N_CLS], lp16[:N_NODES, :N_CLS])
